# Initial kernel scaffold; baseline (speedup 1.0000x reference)
import jax
import jax.numpy as jnp
from jax.experimental import pallas as pl


def kernel(inputs, field_index, w, V):
    raise NotImplementedError("write your pallas kernel here")


# speedup vs baseline: 9.1285x; 9.1285x over previous
import jax
import jax.numpy as jnp
from jax.experimental import pallas as pl

_BATCH = 16384
_F = 100
_NFIELD = 26
_K = 128
_BK = 2048


def _fm_kernel(x_ref, fi_ref, w_ref, v_ref, o_ref):
    fi = fi_ref[0, :]
    onehot = (fi[:, None] ==
              jax.lax.broadcasted_iota(jnp.int32, (_F, _NFIELD), 1)
              ).astype(jnp.float32)
    e = jnp.dot(onehot, v_ref[...],
                preferred_element_type=jnp.float32)
    r = jnp.sum(e * e, axis=1, keepdims=True)
    wv = w_ref[0, :][:, None]

    x = x_ref[...]
    s = jnp.dot(x, e, preferred_element_type=jnp.float32)
    t = jnp.sum(s * s, axis=1, keepdims=True)
    lin = jnp.dot(x, wv, preferred_element_type=jnp.float32)
    u = jnp.dot(x * x, r, preferred_element_type=jnp.float32)
    o_ref[...] = lin + 0.5 * (t - u)


def kernel(inputs, field_index, w, V):
    fi2 = field_index.reshape(1, _F).astype(jnp.int32)
    w2 = w.reshape(1, _F)
    grid = (_BATCH // _BK,)
    out = pl.pallas_call(
        _fm_kernel,
        grid=grid,
        in_specs=[
            pl.BlockSpec((_BK, _F), lambda i: (i, 0)),
            pl.BlockSpec((1, _F), lambda i: (0, 0)),
            pl.BlockSpec((1, _F), lambda i: (0, 0)),
            pl.BlockSpec((_NFIELD, _K), lambda i: (0, 0)),
        ],
        out_specs=pl.BlockSpec((_BK, 1), lambda i: (i, 0)),
        out_shape=jax.ShapeDtypeStruct((_BATCH, 1), jnp.float32),
    )(inputs, fi2, w2, V)
    return out


# speedup vs baseline: 10.0545x; 1.1014x over previous
import jax
import jax.numpy as jnp
from jax.experimental import pallas as pl
from jax.experimental.pallas import tpu as pltpu

_BATCH = 16384
_F = 100
_NFIELD = 26
_K = 128
_BK = 4096


def _fm_kernel(x_ref, fi_ref, w_ref, v_ref, o_ref):
    fi = fi_ref[0, :]
    onehot = (fi[:, None] ==
              jax.lax.broadcasted_iota(jnp.int32, (_F, _NFIELD), 1)
              ).astype(jnp.float32)
    e = jnp.dot(onehot, v_ref[...],
                preferred_element_type=jnp.float32)
    r = jnp.sum(e * e, axis=1, keepdims=True)
    wv = w_ref[0, :][:, None]

    x = x_ref[...]
    s = jnp.dot(x, e, preferred_element_type=jnp.float32)
    t = jnp.sum(s * s, axis=1, keepdims=True)
    lin = jnp.dot(x, wv, preferred_element_type=jnp.float32)
    u = jnp.dot(x * x, r, preferred_element_type=jnp.float32)
    o_ref[...] = lin + 0.5 * (t - u)


def kernel(inputs, field_index, w, V):
    fi2 = field_index.reshape(1, _F).astype(jnp.int32)
    w2 = w.reshape(1, _F)
    grid = (_BATCH // _BK,)
    out = pl.pallas_call(
        _fm_kernel,
        grid=grid,
        in_specs=[
            pl.BlockSpec((_BK, _F), lambda i: (i, 0)),
            pl.BlockSpec((1, _F), lambda i: (0, 0)),
            pl.BlockSpec((1, _F), lambda i: (0, 0)),
            pl.BlockSpec((_NFIELD, _K), lambda i: (0, 0)),
        ],
        out_specs=pl.BlockSpec((_BK, 1), lambda i: (i, 0)),
        out_shape=jax.ShapeDtypeStruct((_BATCH, 1), jnp.float32),
        compiler_params=pltpu.CompilerParams(
            dimension_semantics=("parallel",)),
    )(inputs, fi2, w2, V)
    return out
